# baseline (speedup 1.0000x reference)
import functools

import jax
import jax.numpy as jnp
from jax import lax
from jax.experimental import pallas as pl
from jax.experimental.pallas import tpu as pltpu
from jax.experimental.pallas import tpu_sc as plsc

_N = 50000
_E = 800000
_B = 8
_IN = 60
_EMB = 200
_IN_P = 64
_EMB_P = 208
_LANES = 16
_NSUB = 16
_EBLK = 128
_NBLK = _E // _EBLK
_RPT = 3128
_RPT_LAST = _N - 15 * _RPT
_ZBLKS = 25


def _make_segsum(C):
  mesh = plsc.VectorSubcoreMesh(core_axis_name="c", subcore_axis_name="s")

  @functools.partial(
      pl.kernel,
      out_type=jax.ShapeDtypeStruct((C * _N, _LANES), jnp.float32),
      mesh=mesh,
      scratch_types=[
          pltpu.VMEM((_EBLK,), jnp.int32),
          pltpu.VMEM((_EBLK,), jnp.int32),
          pltpu.VMEM((_EBLK, _LANES), jnp.float32),
          pltpu.VMEM((_EBLK, _LANES), jnp.float32),
          pltpu.VMEM_SHARED((_N, _LANES), jnp.float32),
          pltpu.SemaphoreType.DMA,
      ],
      compiler_params=pltpu.CompilerParams(use_tc_tiling_on_sc=False),
  )
  def segsum(table, src, dst, out, src_v, dst_v, rows_v, zeros_v, acc, sem):
    cid = lax.axis_index("c")
    sid = lax.axis_index("s")

    def zinit(i, carry):
      zeros_v[i, :] = jnp.zeros((_LANES,), jnp.float32)
      return carry

    lax.fori_loop(0, _EBLK, zinit, 0)

    nblk_t = 390 + jnp.where(sid < _NBLK - 390 * _NSUB, 1, 0)
    nck = (C - cid + 1) // 2
    row0 = sid * _RPT

    def chunk_body(k, carry):
      c = 2 * k + cid
      coff = c * _N

      def zblk(j, zc):
        start = jnp.minimum(row0 + j * _EBLK, _N - _EBLK)
        pltpu.sync_copy(zeros_v, acc.at[pl.ds(start, _EBLK)])
        return zc

      lax.fori_loop(0, _ZBLKS, zblk, 0)
      plsc.subcore_barrier()

      def eblk(t, ec):
        off = (sid + _NSUB * t) * _EBLK
        pltpu.sync_copy(src.at[pl.ds(off, _EBLK)], src_v)
        pltpu.sync_copy(dst.at[pl.ds(off, _EBLK)], dst_v)
        for j in range(_EBLK // _LANES):
          sl = pl.ds(j * _LANES, _LANES)
          src_v[sl] = src_v[sl] + coff
        pltpu.async_copy(table.at[src_v], rows_v, sem).wait()
        pltpu.sync_copy(rows_v, acc.at[dst_v], add=True)
        return ec

      lax.fori_loop(0, nblk_t, eblk, 0)
      plsc.subcore_barrier()

      @pl.when(sid < _NSUB - 1)
      def _():
        pltpu.sync_copy(
            acc.at[pl.ds(row0, _RPT)], out.at[pl.ds(coff + row0, _RPT)]
        )

      @pl.when(sid == _NSUB - 1)
      def _():
        pltpu.sync_copy(
            acc.at[pl.ds(row0, _RPT_LAST)],
            out.at[pl.ds(coff + row0, _RPT_LAST)],
        )

      return carry

    lax.fori_loop(0, nck, chunk_body, 0)

  return segsum


_segsum0 = _make_segsum(_IN_P // _LANES)
_segsum1 = _make_segsum(_EMB_P // _LANES)


def _chunked(x):
  n, d = x.shape
  return x.reshape(n, d // _LANES, _LANES).transpose(1, 0, 2).reshape(-1, _LANES)


def _unchunked(t, d):
  return t.reshape(d // _LANES, _N, _LANES).transpose(1, 0, 2).reshape(_N, d)


_BN = 400


def _dense_layer(agg, x, w_rel, w_root, b, relu):
  n, din = x.shape
  dout = w_rel.shape[0]

  def body(agg_ref, x_ref, wr_ref, wo_ref, b_ref, o_ref):
    z = lax.dot_general(agg_ref[...], wr_ref[...], (((1,), (1,)), ((), ())),
                        preferred_element_type=jnp.float32)
    z = z + lax.dot_general(x_ref[...], wo_ref[...], (((1,), (1,)), ((), ())),
                            preferred_element_type=jnp.float32)
    z = z + b_ref[...]
    if relu:
      z = jnp.maximum(z, 0.0)
    o_ref[...] = z

  return pl.pallas_call(
      body,
      grid=(n // _BN,),
      in_specs=[
          pl.BlockSpec((_BN, din), lambda i: (i, 0)),
          pl.BlockSpec((_BN, din), lambda i: (i, 0)),
          pl.BlockSpec((dout, din), lambda i: (0, 0)),
          pl.BlockSpec((dout, din), lambda i: (0, 0)),
          pl.BlockSpec((1, dout), lambda i: (0, 0)),
      ],
      out_specs=pl.BlockSpec((_BN, dout), lambda i: (i, 0)),
      out_shape=jax.ShapeDtypeStruct((n, dout), jnp.float32),
  )(agg, x, w_rel, w_root, b)


def _mut_update(h, hprev, midx, w, b):
  n, p = h.shape
  din = hprev.shape[1]

  def body(idx_ref, hprev_hbm, h_hbm, w_ref, b_ref, out_hbm, mut_ref,
           prev_v, cur_v, new_v, sem):
    for i in range(_B):
      idx = idx_ref[i]
      pltpu.async_copy(hprev_hbm.at[pl.ds(idx, 1)],
                       prev_v.at[pl.ds(i, 1)], sem).wait()
      pltpu.async_copy(h_hbm.at[pl.ds(idx, 1)],
                       cur_v.at[pl.ds(i, 1)], sem).wait()
    z = lax.dot_general(prev_v[...], w_ref[...], (((1,), (1,)), ((), ())),
                        preferred_element_type=jnp.float32)
    z = z + b_ref[...] + cur_v[...]
    new_v[...] = z
    mut_ref[...] = z
    for i in range(_B):
      idx = idx_ref[i]
      pltpu.async_copy(new_v.at[pl.ds(i, 1)],
                       out_hbm.at[pl.ds(idx, 1)], sem).wait()

  return pl.pallas_call(
      body,
      in_specs=[
          pl.BlockSpec(memory_space=pltpu.SMEM),
          pl.BlockSpec(memory_space=pltpu.MemorySpace.HBM),
          pl.BlockSpec(memory_space=pltpu.MemorySpace.HBM),
          pl.BlockSpec((p, din), lambda: (0, 0)),
          pl.BlockSpec((1, p), lambda: (0, 0)),
      ],
      out_specs=[
          pl.BlockSpec(memory_space=pltpu.MemorySpace.HBM),
          pl.BlockSpec((_B, p), lambda: (0, 0)),
      ],
      out_shape=[
          jax.ShapeDtypeStruct((n, p), jnp.float32),
          jax.ShapeDtypeStruct((_B, p), jnp.float32),
      ],
      input_output_aliases={2: 0},
      scratch_shapes=[
          pltpu.VMEM((_B, din), jnp.float32),
          pltpu.VMEM((_B, p), jnp.float32),
          pltpu.VMEM((_B, p), jnp.float32),
          pltpu.SemaphoreType.DMA,
      ],
  )(midx, hprev, h, w, b)


def _pool(h, batch2d):
  n, p = h.shape

  def body(b_ref, h_ref, o_ref):
    @pl.when(pl.program_id(0) == 0)
    def _():
      o_ref[...] = jnp.zeros_like(o_ref)

    onehot = (lax.broadcasted_iota(jnp.int32, (_BN, _B), 1) == b_ref[...]
              ).astype(jnp.float32)
    o_ref[...] += lax.dot_general(onehot, h_ref[...], (((0,), (0,)), ((), ())),
                                  preferred_element_type=jnp.float32)

  return pl.pallas_call(
      body,
      grid=(n // _BN,),
      in_specs=[
          pl.BlockSpec((_BN, 1), lambda i: (i, 0)),
          pl.BlockSpec((_BN, p), lambda i: (i, 0)),
      ],
      out_specs=pl.BlockSpec((_B, p), lambda i: (0, 0)),
      out_shape=jax.ShapeDtypeStruct((_B, p), jnp.float32),
  )(batch2d, h)


def _head(g_be, n_be, g_af, n_af, wa, ba, wb, bb):
  def body(bb_ref, g1, n1, g2, n2, wa_ref, ba_ref, wb_ref, o_ref):
    feat = jnp.concatenate([g1[...], n1[...], g2[...], n2[...]], axis=1)
    z = lax.dot_general(feat, wa_ref[...], (((1,), (1,)), ((), ())),
                        preferred_element_type=jnp.float32) + ba_ref[...]
    z = jnp.where(z > 0, z, 0.1 * z)
    o_ref[...] = lax.dot_general(z, wb_ref[...], (((1,), (1,)), ((), ())),
                                 preferred_element_type=jnp.float32) + bb_ref[0]

  return pl.pallas_call(
      body,
      in_specs=[
          pl.BlockSpec(memory_space=pltpu.SMEM),
          pl.BlockSpec((_B, _EMB_P), lambda: (0, 0)),
          pl.BlockSpec((_B, _EMB_P), lambda: (0, 0)),
          pl.BlockSpec((_B, _EMB_P), lambda: (0, 0)),
          pl.BlockSpec((_B, _EMB_P), lambda: (0, 0)),
          pl.BlockSpec((_EMB_P, 4 * _EMB_P), lambda: (0, 0)),
          pl.BlockSpec((1, _EMB_P), lambda: (0, 0)),
          pl.BlockSpec((128, _EMB_P), lambda: (0, 0)),
      ],
      out_specs=pl.BlockSpec((_B, 128), lambda: (0, 0)),
      out_shape=jax.ShapeDtypeStruct((_B, 128), jnp.float32),
  )(bb, g_be, n_be, g_af, n_af, wa, ba, wb)


def _pad2(w, r, c):
  return jnp.pad(w, ((0, r - w.shape[0]), (0, c - w.shape[1])))


def _offidx(idx, count):
  off = jnp.concatenate(
      [jnp.zeros((1,), count.dtype), jnp.cumsum(count)[:-1]])
  return idx + off


def kernel(x_s, edge_index_s, x_s_batch, wide_res_idx, wt_count, x_t,
           edge_index_t, x_t_batch, mut_res_idx, mut_count, W_rel0, b_rel0,
           W_root0, W_rel1, b_rel1, W_root1, W_fc1, b_fc1, W_fc2, b_fc2,
           W_fca, b_fca, W_fcb, b_fcb):
  w_rel0 = _pad2(W_rel0, _EMB_P, _IN_P)
  w_root0 = _pad2(W_root0, _EMB_P, _IN_P)
  w_fc1 = _pad2(W_fc1, _EMB_P, _IN_P)
  w_rel1 = _pad2(W_rel1, _EMB_P, _EMB_P)
  w_root1 = _pad2(W_root1, _EMB_P, _EMB_P)
  w_fc2 = _pad2(W_fc2, _EMB_P, _EMB_P)
  b_rel0p = jnp.pad(b_rel0, (0, _EMB_P - _EMB)).reshape(1, _EMB_P)
  b_rel1p = jnp.pad(b_rel1, (0, _EMB_P - _EMB)).reshape(1, _EMB_P)
  b_fc1p = jnp.pad(b_fc1, (0, _EMB_P - _EMB)).reshape(1, _EMB_P)
  b_fc2p = jnp.pad(b_fc2, (0, _EMB_P - _EMB)).reshape(1, _EMB_P)
  w_fca = jnp.zeros((_EMB_P, 4 * _EMB_P), jnp.float32)
  for j in range(4):
    w_fca = w_fca.at[:_EMB, j * _EMB_P:j * _EMB_P + _EMB].set(
        W_fca[:, j * _EMB:(j + 1) * _EMB])
  b_fcap = jnp.pad(b_fca, (0, _EMB_P - _EMB)).reshape(1, _EMB_P)
  w_fcb = _pad2(W_fcb, 128, _EMB_P)
  b_fcb2 = b_fcb.reshape(1)

  def one_graph(x, ei, batch, midx):
    xp = jnp.pad(x, ((0, 0), (0, _IN_P - _IN)))
    src = ei[0]
    dst = ei[1]
    agg0 = _unchunked(_segsum0(_chunked(xp), src, dst), _IN_P)
    h1 = _dense_layer(agg0, xp, w_rel0, w_root0, b_rel0p, relu=True)
    h1, _ = _mut_update(h1, xp, midx, w_fc1, b_fc1p)
    agg1 = _unchunked(_segsum1(_chunked(h1), src, dst), _EMB_P)
    h2 = _dense_layer(agg1, h1, w_rel1, w_root1, b_rel1p, relu=False)
    h2, mut = _mut_update(h2, h1, midx, w_fc2, b_fc2p)
    g = _pool(h2, batch.reshape(_N, 1))
    return g, mut

  widx = _offidx(wide_res_idx, wt_count)
  midx = _offidx(mut_res_idx, mut_count)
  g_be, n_be = one_graph(x_s, edge_index_s, x_s_batch, widx)
  g_af, n_af = one_graph(x_t, edge_index_t, x_t_batch, midx)
  out = _head(g_be, n_be, g_af, n_af, w_fca, b_fcap, w_fcb, b_fcb2)
  return out[:, :1]


# speedup vs baseline: 2.8750x; 2.8750x over previous
import functools

import jax
import jax.numpy as jnp
from jax import lax
from jax.experimental import pallas as pl
from jax.experimental.pallas import tpu as pltpu
from jax.experimental.pallas import tpu_sc as plsc

_N = 50000
_E = 800000
_B = 8
_IN = 60
_EMB = 200
_IN_P = 64
_EMB_P = 208
_CW = 16
_LANES = 16
_NSUB = 16
_EBLK = 128
_GRP = 8
_BPT = 392
_NGRP = _BPT // _GRP
_EPAD = _BPT * _NSUB * _EBLK
_NTRASH = 8
_NA = _N + _NTRASH
_RPT = 3128
_RPT_LAST = _N - 15 * _RPT
_ZBLK = 256
_ZBLKS = 13


def _make_segsum(C):
  mesh = plsc.VectorSubcoreMesh(core_axis_name="c", subcore_axis_name="s")

  @functools.partial(
      pl.kernel,
      out_type=jax.ShapeDtypeStruct((C * _N, _CW), jnp.float32),
      mesh=mesh,
      scratch_types=[
          pltpu.VMEM((_GRP, _EBLK), jnp.int32),
          pltpu.VMEM((_GRP, _EBLK), jnp.int32),
          pltpu.VMEM((_GRP, _EBLK, _CW), jnp.float32),
          pltpu.VMEM((_ZBLK, _CW), jnp.float32),
          pltpu.VMEM_SHARED((_NA, _CW), jnp.float32),
          pltpu.SemaphoreType.DMA,
          pltpu.SemaphoreType.DMA,
      ],
      compiler_params=pltpu.CompilerParams(use_tc_tiling_on_sc=False),
  )
  def segsum(table, src2d, dst2d, out, src_g, dst_g, rows_g, zeros_v, acc,
             sem_g, sem_s):
    cid = lax.axis_index("c")
    sid = lax.axis_index("s")

    def zinit(i, carry):
      zeros_v[i, :] = jnp.zeros((_LANES,), jnp.float32)
      return carry

    lax.fori_loop(0, _ZBLK, zinit, 0)

    nck = (C - cid + 1) // 2
    row0 = sid * _RPT
    b0 = sid * _BPT

    def chunk_body(k, carry):
      c = 2 * k + cid
      coff = c * _N

      def zblk(j, zc):
        start = jnp.minimum(row0 + j * _ZBLK, _NA - _ZBLK)
        pltpu.sync_copy(zeros_v, acc.at[pl.ds(start, _ZBLK)])
        return zc

      lax.fori_loop(0, _ZBLKS, zblk, 0)
      plsc.subcore_barrier()

      def grp(g, ec):
        gb0 = b0 + g * _GRP
        pltpu.sync_copy(src2d.at[pl.ds(gb0, _GRP)], src_g)
        pltpu.sync_copy(dst2d.at[pl.ds(gb0, _GRP)], dst_g)
        gds = []
        for j in range(_GRP):
          for q in range(_EBLK // _LANES):
            sl = pl.ds(q * _LANES, _LANES)
            src_g[j, sl] = src_g[j, sl] + coff
          gds.append(
              pltpu.async_copy(table.at[src_g.at[j]], rows_g.at[j], sem_g))
        for d in gds:
          d.wait()
        sds = []
        for j in range(_GRP):
          sds.append(
              pltpu.async_copy(
                  rows_g.at[j], acc.at[dst_g.at[j]], sem_s, add=True))
        for d in sds:
          d.wait()
        return ec

      lax.fori_loop(0, _NGRP, grp, 0)
      plsc.subcore_barrier()

      @pl.when(sid < _NSUB - 1)
      def _():
        pltpu.sync_copy(
            acc.at[pl.ds(row0, _RPT)], out.at[pl.ds(coff + row0, _RPT)]
        )

      @pl.when(sid == _NSUB - 1)
      def _():
        pltpu.sync_copy(
            acc.at[pl.ds(row0, _RPT_LAST)],
            out.at[pl.ds(coff + row0, _RPT_LAST)],
        )

      return carry

    lax.fori_loop(0, nck, chunk_body, 0)

  return segsum


_segsum0 = _make_segsum(_IN_P // _CW)
_segsum1 = _make_segsum(_EMB_P // _CW)


def _chunked(x):
  n, d = x.shape
  return x.reshape(n, d // _CW, _CW).transpose(1, 0, 2).reshape(-1, _CW)


def _unchunked(t, d):
  return t.reshape(d // _CW, _N, _CW).transpose(1, 0, 2).reshape(_N, d)


def _pad_edges(src, dst):
  npad = _EPAD - _E
  src_p = jnp.concatenate([src, src[:npad]]).reshape(-1, _EBLK)
  dst_p = jnp.concatenate(
      [dst, _N + (jnp.arange(npad, dtype=dst.dtype) % _NTRASH)]
  ).reshape(-1, _EBLK)
  return src_p, dst_p


_BN = 400


def _dense_layer(agg, x, w_rel, w_root, b, relu):
  n, din = x.shape
  dout = w_rel.shape[0]

  def body(agg_ref, x_ref, wr_ref, wo_ref, b_ref, o_ref):
    z = lax.dot_general(agg_ref[...], wr_ref[...], (((1,), (1,)), ((), ())),
                        preferred_element_type=jnp.float32)
    z = z + lax.dot_general(x_ref[...], wo_ref[...], (((1,), (1,)), ((), ())),
                            preferred_element_type=jnp.float32)
    z = z + b_ref[...]
    if relu:
      z = jnp.maximum(z, 0.0)
    o_ref[...] = z

  return pl.pallas_call(
      body,
      grid=(n // _BN,),
      in_specs=[
          pl.BlockSpec((_BN, din), lambda i: (i, 0)),
          pl.BlockSpec((_BN, din), lambda i: (i, 0)),
          pl.BlockSpec((dout, din), lambda i: (0, 0)),
          pl.BlockSpec((dout, din), lambda i: (0, 0)),
          pl.BlockSpec((1, dout), lambda i: (0, 0)),
      ],
      out_specs=pl.BlockSpec((_BN, dout), lambda i: (i, 0)),
      out_shape=jax.ShapeDtypeStruct((n, dout), jnp.float32),
  )(agg, x, w_rel, w_root, b)


def _mut_update(h, hprev, midx, w, b):
  n, p = h.shape
  din = hprev.shape[1]

  def body(idx_ref, hprev_hbm, h_hbm, w_ref, b_ref, out_hbm, mut_ref,
           prev_v, cur_v, new_v, sem):
    for i in range(_B):
      idx = idx_ref[i]
      pltpu.async_copy(hprev_hbm.at[pl.ds(idx, 1)],
                       prev_v.at[pl.ds(i, 1)], sem).wait()
      pltpu.async_copy(h_hbm.at[pl.ds(idx, 1)],
                       cur_v.at[pl.ds(i, 1)], sem).wait()
    z = lax.dot_general(prev_v[...], w_ref[...], (((1,), (1,)), ((), ())),
                        preferred_element_type=jnp.float32)
    z = z + b_ref[...] + cur_v[...]
    new_v[...] = z
    mut_ref[...] = z
    for i in range(_B):
      idx = idx_ref[i]
      pltpu.async_copy(new_v.at[pl.ds(i, 1)],
                       out_hbm.at[pl.ds(idx, 1)], sem).wait()

  return pl.pallas_call(
      body,
      in_specs=[
          pl.BlockSpec(memory_space=pltpu.SMEM),
          pl.BlockSpec(memory_space=pltpu.MemorySpace.HBM),
          pl.BlockSpec(memory_space=pltpu.MemorySpace.HBM),
          pl.BlockSpec((p, din), lambda: (0, 0)),
          pl.BlockSpec((1, p), lambda: (0, 0)),
      ],
      out_specs=[
          pl.BlockSpec(memory_space=pltpu.MemorySpace.HBM),
          pl.BlockSpec((_B, p), lambda: (0, 0)),
      ],
      out_shape=[
          jax.ShapeDtypeStruct((n, p), jnp.float32),
          jax.ShapeDtypeStruct((_B, p), jnp.float32),
      ],
      input_output_aliases={2: 0},
      scratch_shapes=[
          pltpu.VMEM((_B, din), jnp.float32),
          pltpu.VMEM((_B, p), jnp.float32),
          pltpu.VMEM((_B, p), jnp.float32),
          pltpu.SemaphoreType.DMA,
      ],
  )(midx, hprev, h, w, b)


def _pool(h, batch2d):
  n, p = h.shape

  def body(b_ref, h_ref, o_ref):
    @pl.when(pl.program_id(0) == 0)
    def _():
      o_ref[...] = jnp.zeros_like(o_ref)

    onehot = (lax.broadcasted_iota(jnp.int32, (_BN, _B), 1) == b_ref[...]
              ).astype(jnp.float32)
    o_ref[...] += lax.dot_general(onehot, h_ref[...], (((0,), (0,)), ((), ())),
                                  preferred_element_type=jnp.float32)

  return pl.pallas_call(
      body,
      grid=(n // _BN,),
      in_specs=[
          pl.BlockSpec((_BN, 1), lambda i: (i, 0)),
          pl.BlockSpec((_BN, p), lambda i: (i, 0)),
      ],
      out_specs=pl.BlockSpec((_B, p), lambda i: (0, 0)),
      out_shape=jax.ShapeDtypeStruct((_B, p), jnp.float32),
  )(batch2d, h)


def _head(g_be, n_be, g_af, n_af, wa, ba, wb, bb):
  def body(bb_ref, g1, n1, g2, n2, wa_ref, ba_ref, wb_ref, o_ref):
    feat = jnp.concatenate([g1[...], n1[...], g2[...], n2[...]], axis=1)
    z = lax.dot_general(feat, wa_ref[...], (((1,), (1,)), ((), ())),
                        preferred_element_type=jnp.float32) + ba_ref[...]
    z = jnp.where(z > 0, z, 0.1 * z)
    o_ref[...] = lax.dot_general(z, wb_ref[...], (((1,), (1,)), ((), ())),
                                 preferred_element_type=jnp.float32) + bb_ref[0]

  return pl.pallas_call(
      body,
      in_specs=[
          pl.BlockSpec(memory_space=pltpu.SMEM),
          pl.BlockSpec((_B, _EMB_P), lambda: (0, 0)),
          pl.BlockSpec((_B, _EMB_P), lambda: (0, 0)),
          pl.BlockSpec((_B, _EMB_P), lambda: (0, 0)),
          pl.BlockSpec((_B, _EMB_P), lambda: (0, 0)),
          pl.BlockSpec((_EMB_P, 4 * _EMB_P), lambda: (0, 0)),
          pl.BlockSpec((1, _EMB_P), lambda: (0, 0)),
          pl.BlockSpec((128, _EMB_P), lambda: (0, 0)),
      ],
      out_specs=pl.BlockSpec((_B, 128), lambda: (0, 0)),
      out_shape=jax.ShapeDtypeStruct((_B, 128), jnp.float32),
  )(bb, g_be, n_be, g_af, n_af, wa, ba, wb)


def _pad2(w, r, c):
  return jnp.pad(w, ((0, r - w.shape[0]), (0, c - w.shape[1])))


def _offidx(idx, count):
  off = jnp.concatenate(
      [jnp.zeros((1,), count.dtype), jnp.cumsum(count)[:-1]])
  return idx + off


def kernel(x_s, edge_index_s, x_s_batch, wide_res_idx, wt_count, x_t,
           edge_index_t, x_t_batch, mut_res_idx, mut_count, W_rel0, b_rel0,
           W_root0, W_rel1, b_rel1, W_root1, W_fc1, b_fc1, W_fc2, b_fc2,
           W_fca, b_fca, W_fcb, b_fcb):
  w_rel0 = _pad2(W_rel0, _EMB_P, _IN_P)
  w_root0 = _pad2(W_root0, _EMB_P, _IN_P)
  w_fc1 = _pad2(W_fc1, _EMB_P, _IN_P)
  w_rel1 = _pad2(W_rel1, _EMB_P, _EMB_P)
  w_root1 = _pad2(W_root1, _EMB_P, _EMB_P)
  w_fc2 = _pad2(W_fc2, _EMB_P, _EMB_P)
  b_rel0p = jnp.pad(b_rel0, (0, _EMB_P - _EMB)).reshape(1, _EMB_P)
  b_rel1p = jnp.pad(b_rel1, (0, _EMB_P - _EMB)).reshape(1, _EMB_P)
  b_fc1p = jnp.pad(b_fc1, (0, _EMB_P - _EMB)).reshape(1, _EMB_P)
  b_fc2p = jnp.pad(b_fc2, (0, _EMB_P - _EMB)).reshape(1, _EMB_P)
  w_fca = jnp.zeros((_EMB_P, 4 * _EMB_P), jnp.float32)
  for j in range(4):
    w_fca = w_fca.at[:_EMB, j * _EMB_P:j * _EMB_P + _EMB].set(
        W_fca[:, j * _EMB:(j + 1) * _EMB])
  b_fcap = jnp.pad(b_fca, (0, _EMB_P - _EMB)).reshape(1, _EMB_P)
  w_fcb = _pad2(W_fcb, 128, _EMB_P)
  b_fcb2 = b_fcb.reshape(1)

  def one_graph(x, ei, batch, midx):
    xp = jnp.pad(x, ((0, 0), (0, _IN_P - _IN)))
    src2d, dst2d = _pad_edges(ei[0], ei[1])
    agg0 = _unchunked(_segsum0(_chunked(xp), src2d, dst2d), _IN_P)
    h1 = _dense_layer(agg0, xp, w_rel0, w_root0, b_rel0p, relu=True)
    h1, _ = _mut_update(h1, xp, midx, w_fc1, b_fc1p)
    agg1 = _unchunked(_segsum1(_chunked(h1), src2d, dst2d), _EMB_P)
    h2 = _dense_layer(agg1, h1, w_rel1, w_root1, b_rel1p, relu=False)
    h2, mut = _mut_update(h2, h1, midx, w_fc2, b_fc2p)
    g = _pool(h2, batch.reshape(_N, 1))
    return g, mut

  widx = _offidx(wide_res_idx, wt_count)
  midx = _offidx(mut_res_idx, mut_count)
  g_be, n_be = one_graph(x_s, edge_index_s, x_s_batch, widx)
  g_af, n_af = one_graph(x_t, edge_index_t, x_t_batch, midx)
  out = _head(g_be, n_be, g_af, n_af, w_fca, b_fcap, w_fcb, b_fcb2)
  return out[:, :1]


# speedup vs baseline: 4.3805x; 1.5237x over previous
import functools

import jax
import jax.numpy as jnp
from jax import lax
from jax.experimental import pallas as pl
from jax.experimental.pallas import tpu as pltpu
from jax.experimental.pallas import tpu_sc as plsc

_N = 50000
_E = 800000
_B = 8
_IN = 60
_EMB = 200
_IN_P = 64
_EMB_P = 208
_CW = 16
_LANES = 16
_NSUB = 16
_EBLK = 128
_GRP = 14
_BPT = 392
_NGRP = _BPT // _GRP
_EPAD = _BPT * _NSUB * _EBLK
_NTRASH = 8
_NA = _N + _NTRASH
_RPT = 3128
_RPT_LAST = _N - 15 * _RPT
_ZBLK = 256
_ZBLKS = 13


def _make_segsum(C):
  mesh = plsc.VectorSubcoreMesh(core_axis_name="c", subcore_axis_name="s")

  @functools.partial(
      pl.kernel,
      out_type=jax.ShapeDtypeStruct((C * _N, _CW), jnp.float32),
      mesh=mesh,
      scratch_types=[
          pltpu.VMEM((_GRP, _EBLK), jnp.int32),
          pltpu.VMEM((_GRP, _EBLK), jnp.int32),
          pltpu.VMEM((_GRP, _EBLK, _CW), jnp.float32),
          pltpu.VMEM((_GRP, _EBLK), jnp.int32),
          pltpu.VMEM((_GRP, _EBLK), jnp.int32),
          pltpu.VMEM((_GRP, _EBLK, _CW), jnp.float32),
          pltpu.VMEM((_ZBLK, _CW), jnp.float32),
          pltpu.VMEM_SHARED((_NA, _CW), jnp.float32),
          pltpu.SemaphoreType.DMA,
          pltpu.SemaphoreType.DMA,
          pltpu.SemaphoreType.DMA,
          pltpu.SemaphoreType.DMA,
      ],
      compiler_params=pltpu.CompilerParams(use_tc_tiling_on_sc=False),
  )
  def segsum(table, src2d, dst2d, out, src_g0, dst_g0, rows_g0, src_g1,
             dst_g1, rows_g1, zeros_v, acc, sem_g0, sem_g1, sem_s0, sem_s1):
    cid = lax.axis_index("c")
    sid = lax.axis_index("s")

    def zinit(i, carry):
      zeros_v[i, :] = jnp.zeros((_LANES,), jnp.float32)
      return carry

    lax.fori_loop(0, _ZBLK, zinit, 0)

    nck = (C - cid + 1) // 2
    row0 = sid * _RPT
    b0 = sid * _BPT

    def chunk_body(k, carry):
      c = 2 * k + cid
      coff = c * _N

      def zblk(j, zc):
        start = jnp.minimum(row0 + j * _ZBLK, _NA - _ZBLK)
        pltpu.sync_copy(zeros_v, acc.at[pl.ds(start, _ZBLK)])
        return zc

      lax.fori_loop(0, _ZBLKS, zblk, 0)
      plsc.subcore_barrier()

      tab_c = table.at[pl.ds(coff, _N)]
      bufs = ((src_g0, dst_g0, rows_g0, sem_g0, sem_s0),
              (src_g1, dst_g1, rows_g1, sem_g1, sem_s1))

      def load_fire(g, bi):
        src_g, dst_g, rows_g, sem_g, _ = bufs[bi]
        gb0 = b0 + g * _GRP
        pltpu.sync_copy(src2d.at[pl.ds(gb0, _GRP)], src_g)
        pltpu.sync_copy(dst2d.at[pl.ds(gb0, _GRP)], dst_g)
        for j in range(_GRP):
          pltpu.async_copy(tab_c.at[src_g.at[j]], rows_g.at[j], sem_g)

      def wait_gathers(bi):
        src_g, _, rows_g, sem_g, _ = bufs[bi]
        for j in range(_GRP):
          pltpu.make_async_copy(
              tab_c.at[src_g.at[j]], rows_g.at[j], sem_g).wait()

      def fire_scatters(bi):
        _, dst_g, rows_g, _, sem_s = bufs[bi]
        for j in range(_GRP):
          pltpu.async_copy(rows_g.at[j], acc.at[dst_g.at[j]], sem_s, add=True)

      def wait_scatters(bi):
        _, dst_g, rows_g, _, sem_s = bufs[bi]
        for j in range(_GRP):
          pltpu.make_async_copy(rows_g.at[j], acc.at[dst_g.at[j]], sem_s).wait()

      load_fire(0, 0)
      load_fire(1, 1)

      def pipe(k2, pc):
        for bi in range(2):
          g = 2 * k2 + bi

          wait_gathers(bi)
          fire_scatters(bi)

          @pl.when(g + 2 < _NGRP)
          def _():
            wait_scatters(bi)
            load_fire(g + 2, bi)

        return pc

      lax.fori_loop(0, _NGRP // 2, pipe, 0)
      wait_scatters(0)
      wait_scatters(1)
      plsc.subcore_barrier()

      @pl.when(sid < _NSUB - 1)
      def _():
        pltpu.sync_copy(
            acc.at[pl.ds(row0, _RPT)], out.at[pl.ds(coff + row0, _RPT)]
        )

      @pl.when(sid == _NSUB - 1)
      def _():
        pltpu.sync_copy(
            acc.at[pl.ds(row0, _RPT_LAST)],
            out.at[pl.ds(coff + row0, _RPT_LAST)],
        )

      return carry

    lax.fori_loop(0, nck, chunk_body, 0)

  return segsum


_segsum0 = _make_segsum(_IN_P // _CW)
_segsum1 = _make_segsum(_EMB_P // _CW)


def _chunked(x):
  n, d = x.shape
  return x.reshape(n, d // _CW, _CW).transpose(1, 0, 2).reshape(-1, _CW)


def _unchunked(t, d):
  return t.reshape(d // _CW, _N, _CW).transpose(1, 0, 2).reshape(_N, d)


def _pad_edges(src, dst):
  npad = _EPAD - _E
  src_p = jnp.concatenate([src, src[:npad]]).reshape(-1, _EBLK)
  dst_p = jnp.concatenate(
      [dst, _N + (jnp.arange(npad, dtype=dst.dtype) % _NTRASH)]
  ).reshape(-1, _EBLK)
  return src_p, dst_p


_BN = 400


def _dense_layer(agg, x, w_rel, w_root, b, relu):
  n, din = x.shape
  dout = w_rel.shape[0]

  def body(agg_ref, x_ref, wr_ref, wo_ref, b_ref, o_ref):
    z = lax.dot_general(agg_ref[...], wr_ref[...], (((1,), (1,)), ((), ())),
                        preferred_element_type=jnp.float32)
    z = z + lax.dot_general(x_ref[...], wo_ref[...], (((1,), (1,)), ((), ())),
                            preferred_element_type=jnp.float32)
    z = z + b_ref[...]
    if relu:
      z = jnp.maximum(z, 0.0)
    o_ref[...] = z

  return pl.pallas_call(
      body,
      grid=(n // _BN,),
      in_specs=[
          pl.BlockSpec((_BN, din), lambda i: (i, 0)),
          pl.BlockSpec((_BN, din), lambda i: (i, 0)),
          pl.BlockSpec((dout, din), lambda i: (0, 0)),
          pl.BlockSpec((dout, din), lambda i: (0, 0)),
          pl.BlockSpec((1, dout), lambda i: (0, 0)),
      ],
      out_specs=pl.BlockSpec((_BN, dout), lambda i: (i, 0)),
      out_shape=jax.ShapeDtypeStruct((n, dout), jnp.float32),
  )(agg, x, w_rel, w_root, b)


def _mut_update(h, hprev, midx, w, b):
  n, p = h.shape
  din = hprev.shape[1]

  def body(idx_ref, hprev_hbm, h_hbm, w_ref, b_ref, out_hbm, mut_ref,
           prev_v, cur_v, new_v, sem):
    for i in range(_B):
      idx = idx_ref[i]
      pltpu.async_copy(hprev_hbm.at[pl.ds(idx, 1)],
                       prev_v.at[pl.ds(i, 1)], sem).wait()
      pltpu.async_copy(h_hbm.at[pl.ds(idx, 1)],
                       cur_v.at[pl.ds(i, 1)], sem).wait()
    z = lax.dot_general(prev_v[...], w_ref[...], (((1,), (1,)), ((), ())),
                        preferred_element_type=jnp.float32)
    z = z + b_ref[...] + cur_v[...]
    new_v[...] = z
    mut_ref[...] = z
    for i in range(_B):
      idx = idx_ref[i]
      pltpu.async_copy(new_v.at[pl.ds(i, 1)],
                       out_hbm.at[pl.ds(idx, 1)], sem).wait()

  return pl.pallas_call(
      body,
      in_specs=[
          pl.BlockSpec(memory_space=pltpu.SMEM),
          pl.BlockSpec(memory_space=pltpu.MemorySpace.HBM),
          pl.BlockSpec(memory_space=pltpu.MemorySpace.HBM),
          pl.BlockSpec((p, din), lambda: (0, 0)),
          pl.BlockSpec((1, p), lambda: (0, 0)),
      ],
      out_specs=[
          pl.BlockSpec(memory_space=pltpu.MemorySpace.HBM),
          pl.BlockSpec((_B, p), lambda: (0, 0)),
      ],
      out_shape=[
          jax.ShapeDtypeStruct((n, p), jnp.float32),
          jax.ShapeDtypeStruct((_B, p), jnp.float32),
      ],
      input_output_aliases={2: 0},
      scratch_shapes=[
          pltpu.VMEM((_B, din), jnp.float32),
          pltpu.VMEM((_B, p), jnp.float32),
          pltpu.VMEM((_B, p), jnp.float32),
          pltpu.SemaphoreType.DMA,
      ],
  )(midx, hprev, h, w, b)


def _pool(h, batch2d):
  n, p = h.shape

  def body(b_ref, h_ref, o_ref):
    @pl.when(pl.program_id(0) == 0)
    def _():
      o_ref[...] = jnp.zeros_like(o_ref)

    onehot = (lax.broadcasted_iota(jnp.int32, (_BN, _B), 1) == b_ref[...]
              ).astype(jnp.float32)
    o_ref[...] += lax.dot_general(onehot, h_ref[...], (((0,), (0,)), ((), ())),
                                  preferred_element_type=jnp.float32)

  return pl.pallas_call(
      body,
      grid=(n // _BN,),
      in_specs=[
          pl.BlockSpec((_BN, 1), lambda i: (i, 0)),
          pl.BlockSpec((_BN, p), lambda i: (i, 0)),
      ],
      out_specs=pl.BlockSpec((_B, p), lambda i: (0, 0)),
      out_shape=jax.ShapeDtypeStruct((_B, p), jnp.float32),
  )(batch2d, h)


def _head(g_be, n_be, g_af, n_af, wa, ba, wb, bb):
  def body(bb_ref, g1, n1, g2, n2, wa_ref, ba_ref, wb_ref, o_ref):
    feat = jnp.concatenate([g1[...], n1[...], g2[...], n2[...]], axis=1)
    z = lax.dot_general(feat, wa_ref[...], (((1,), (1,)), ((), ())),
                        preferred_element_type=jnp.float32) + ba_ref[...]
    z = jnp.where(z > 0, z, 0.1 * z)
    o_ref[...] = lax.dot_general(z, wb_ref[...], (((1,), (1,)), ((), ())),
                                 preferred_element_type=jnp.float32) + bb_ref[0]

  return pl.pallas_call(
      body,
      in_specs=[
          pl.BlockSpec(memory_space=pltpu.SMEM),
          pl.BlockSpec((_B, _EMB_P), lambda: (0, 0)),
          pl.BlockSpec((_B, _EMB_P), lambda: (0, 0)),
          pl.BlockSpec((_B, _EMB_P), lambda: (0, 0)),
          pl.BlockSpec((_B, _EMB_P), lambda: (0, 0)),
          pl.BlockSpec((_EMB_P, 4 * _EMB_P), lambda: (0, 0)),
          pl.BlockSpec((1, _EMB_P), lambda: (0, 0)),
          pl.BlockSpec((128, _EMB_P), lambda: (0, 0)),
      ],
      out_specs=pl.BlockSpec((_B, 128), lambda: (0, 0)),
      out_shape=jax.ShapeDtypeStruct((_B, 128), jnp.float32),
  )(bb, g_be, n_be, g_af, n_af, wa, ba, wb)


def _pad2(w, r, c):
  return jnp.pad(w, ((0, r - w.shape[0]), (0, c - w.shape[1])))


def _offidx(idx, count):
  off = jnp.concatenate(
      [jnp.zeros((1,), count.dtype), jnp.cumsum(count)[:-1]])
  return idx + off


def kernel(x_s, edge_index_s, x_s_batch, wide_res_idx, wt_count, x_t,
           edge_index_t, x_t_batch, mut_res_idx, mut_count, W_rel0, b_rel0,
           W_root0, W_rel1, b_rel1, W_root1, W_fc1, b_fc1, W_fc2, b_fc2,
           W_fca, b_fca, W_fcb, b_fcb):
  w_rel0 = _pad2(W_rel0, _EMB_P, _IN_P)
  w_root0 = _pad2(W_root0, _EMB_P, _IN_P)
  w_fc1 = _pad2(W_fc1, _EMB_P, _IN_P)
  w_rel1 = _pad2(W_rel1, _EMB_P, _EMB_P)
  w_root1 = _pad2(W_root1, _EMB_P, _EMB_P)
  w_fc2 = _pad2(W_fc2, _EMB_P, _EMB_P)
  b_rel0p = jnp.pad(b_rel0, (0, _EMB_P - _EMB)).reshape(1, _EMB_P)
  b_rel1p = jnp.pad(b_rel1, (0, _EMB_P - _EMB)).reshape(1, _EMB_P)
  b_fc1p = jnp.pad(b_fc1, (0, _EMB_P - _EMB)).reshape(1, _EMB_P)
  b_fc2p = jnp.pad(b_fc2, (0, _EMB_P - _EMB)).reshape(1, _EMB_P)
  w_fca = jnp.zeros((_EMB_P, 4 * _EMB_P), jnp.float32)
  for j in range(4):
    w_fca = w_fca.at[:_EMB, j * _EMB_P:j * _EMB_P + _EMB].set(
        W_fca[:, j * _EMB:(j + 1) * _EMB])
  b_fcap = jnp.pad(b_fca, (0, _EMB_P - _EMB)).reshape(1, _EMB_P)
  w_fcb = _pad2(W_fcb, 128, _EMB_P)
  b_fcb2 = b_fcb.reshape(1)

  def one_graph(x, ei, batch, midx):
    xp = jnp.pad(x, ((0, 0), (0, _IN_P - _IN)))
    src2d, dst2d = _pad_edges(ei[0], ei[1])
    agg0 = _unchunked(_segsum0(_chunked(xp), src2d, dst2d), _IN_P)
    h1 = _dense_layer(agg0, xp, w_rel0, w_root0, b_rel0p, relu=True)
    h1, _ = _mut_update(h1, xp, midx, w_fc1, b_fc1p)
    agg1 = _unchunked(_segsum1(_chunked(h1), src2d, dst2d), _EMB_P)
    h2 = _dense_layer(agg1, h1, w_rel1, w_root1, b_rel1p, relu=False)
    h2, mut = _mut_update(h2, h1, midx, w_fc2, b_fc2p)
    g = _pool(h2, batch.reshape(_N, 1))
    return g, mut

  widx = _offidx(wide_res_idx, wt_count)
  midx = _offidx(mut_res_idx, mut_count)
  g_be, n_be = one_graph(x_s, edge_index_s, x_s_batch, widx)
  g_af, n_af = one_graph(x_t, edge_index_t, x_t_batch, midx)
  out = _head(g_be, n_be, g_af, n_af, w_fca, b_fcap, w_fcb, b_fcb2)
  return out[:, :1]


# speedup vs baseline: 5.0660x; 1.1565x over previous
import functools

import jax
import jax.numpy as jnp
from jax import lax
from jax.experimental import pallas as pl
from jax.experimental.pallas import tpu as pltpu
from jax.experimental.pallas import tpu_sc as plsc

_N = 50000
_E = 800000
_B = 8
_IN = 60
_EMB = 200
_IN_P = 64
_EMB_P = 208
_CW = 16
_LANES = 16
_NSUB = 16
_EBLK = 128
_GRP = 14
_BPT = 392
_NGRP = _BPT // _GRP
_EPAD = _BPT * _NSUB * _EBLK
_NTRASH = 8
_NA = _N + _NTRASH
_RPT = 3128
_RPT_LAST = _N - 15 * _RPT
_ZBLK = 256
_ZBLKS = 13


def _make_segsum(C):
  mesh = plsc.VectorSubcoreMesh(core_axis_name="c", subcore_axis_name="s")

  @functools.partial(
      pl.kernel,
      out_type=jax.ShapeDtypeStruct((C * _N, _CW), jnp.float32),
      mesh=mesh,
      scratch_types=[
          pltpu.VMEM((_GRP, _EBLK), jnp.int32),
          pltpu.VMEM((_GRP, _EBLK), jnp.int32),
          pltpu.VMEM((_GRP, _EBLK, _CW), jnp.float32),
          pltpu.VMEM((_GRP, _EBLK), jnp.int32),
          pltpu.VMEM((_GRP, _EBLK), jnp.int32),
          pltpu.VMEM((_GRP, _EBLK, _CW), jnp.float32),
          pltpu.VMEM((_ZBLK, _CW), jnp.float32),
          pltpu.VMEM_SHARED((_NA, _CW), jnp.float32),
          pltpu.SemaphoreType.DMA,
          pltpu.SemaphoreType.DMA,
          pltpu.SemaphoreType.DMA,
          pltpu.SemaphoreType.DMA,
      ],
      compiler_params=pltpu.CompilerParams(use_tc_tiling_on_sc=False),
  )
  def segsum(table, src2d, dst2d, out, src_g0, dst_g0, rows_g0, src_g1,
             dst_g1, rows_g1, zeros_v, acc, sem_g0, sem_g1, sem_s0, sem_s1):
    cid = lax.axis_index("c")
    sid = lax.axis_index("s")

    def zinit(i, carry):
      zeros_v[i, :] = jnp.zeros((_LANES,), jnp.float32)
      return carry

    lax.fori_loop(0, _ZBLK, zinit, 0)

    nck = (C - cid + 1) // 2
    row0 = sid * _RPT
    b0 = sid * _BPT

    def chunk_body(k, carry):
      c = 2 * k + cid

      def zblk(j, zc):
        start = jnp.minimum(row0 + j * _ZBLK, _NA - _ZBLK)
        pltpu.sync_copy(zeros_v, acc.at[pl.ds(start, _ZBLK)])
        return zc

      lax.fori_loop(0, _ZBLKS, zblk, 0)
      plsc.subcore_barrier()

      bufs = ((src_g0, dst_g0, rows_g0, sem_g0, sem_s0),
              (src_g1, dst_g1, rows_g1, sem_g1, sem_s1))

      def load_fire(g, bi):
        src_g, dst_g, rows_g, sem_g, _ = bufs[bi]
        gb0 = b0 + g * _GRP
        pltpu.sync_copy(src2d.at[pl.ds(gb0, _GRP)], src_g)
        pltpu.sync_copy(dst2d.at[pl.ds(gb0, _GRP)], dst_g)
        for j in range(_GRP):
          for q in range(_EBLK // _LANES):
            sl = pl.ds(q * _LANES, _LANES)
            src_g[j, sl] = src_g[j, sl] + c
          pltpu.async_copy(table.at[src_g.at[j]], rows_g.at[j], sem_g)

      def wait_gathers(bi):
        src_g, _, rows_g, sem_g, _ = bufs[bi]
        for j in range(_GRP):
          pltpu.make_async_copy(
              table.at[src_g.at[j]], rows_g.at[j], sem_g).wait()

      def fire_scatters(bi):
        _, dst_g, rows_g, _, sem_s = bufs[bi]
        for j in range(_GRP):
          pltpu.async_copy(rows_g.at[j], acc.at[dst_g.at[j]], sem_s, add=True)

      def wait_scatters(bi):
        _, dst_g, rows_g, _, sem_s = bufs[bi]
        for j in range(_GRP):
          pltpu.make_async_copy(rows_g.at[j], acc.at[dst_g.at[j]], sem_s).wait()

      load_fire(0, 0)
      load_fire(1, 1)

      def pipe(k2, pc):
        for bi in range(2):
          g = 2 * k2 + bi

          wait_gathers(bi)
          fire_scatters(bi)

          @pl.when(g + 2 < _NGRP)
          def _():
            wait_scatters(bi)
            load_fire(g + 2, bi)

        return pc

      lax.fori_loop(0, _NGRP // 2, pipe, 0)
      wait_scatters(0)
      wait_scatters(1)
      plsc.subcore_barrier()

      coff = c * _N

      @pl.when(sid < _NSUB - 1)
      def _():
        pltpu.sync_copy(
            acc.at[pl.ds(row0, _RPT)], out.at[pl.ds(coff + row0, _RPT)]
        )

      @pl.when(sid == _NSUB - 1)
      def _():
        pltpu.sync_copy(
            acc.at[pl.ds(row0, _RPT_LAST)],
            out.at[pl.ds(coff + row0, _RPT_LAST)],
        )

      return carry

    lax.fori_loop(0, nck, chunk_body, 0)

  return segsum


_segsum0 = _make_segsum(_IN_P // _CW)
_segsum1 = _make_segsum(_EMB_P // _CW)


def _unchunked(t, d):
  return t.reshape(d // _CW, _N, _CW).transpose(1, 0, 2).reshape(_N, d)


def _pad_src(src, C):
  npad = _EPAD - _E
  s = src * C
  return jnp.concatenate([s, s[:npad]]).reshape(-1, _EBLK)


def _pad_dst(dst):
  npad = _EPAD - _E
  return jnp.concatenate(
      [dst, _N + (jnp.arange(npad, dtype=dst.dtype) % _NTRASH)]
  ).reshape(-1, _EBLK)


_BN = 400


def _dense_layer(agg, x, w_rel, w_root, b, relu):
  n, din = x.shape
  dout = w_rel.shape[0]

  def body(agg_ref, x_ref, wr_ref, wo_ref, b_ref, o_ref):
    z = lax.dot_general(agg_ref[...], wr_ref[...], (((1,), (1,)), ((), ())),
                        preferred_element_type=jnp.float32)
    z = z + lax.dot_general(x_ref[...], wo_ref[...], (((1,), (1,)), ((), ())),
                            preferred_element_type=jnp.float32)
    z = z + b_ref[...]
    if relu:
      z = jnp.maximum(z, 0.0)
    o_ref[...] = z

  return pl.pallas_call(
      body,
      grid=(n // _BN,),
      in_specs=[
          pl.BlockSpec((_BN, din), lambda i: (i, 0)),
          pl.BlockSpec((_BN, din), lambda i: (i, 0)),
          pl.BlockSpec((dout, din), lambda i: (0, 0)),
          pl.BlockSpec((dout, din), lambda i: (0, 0)),
          pl.BlockSpec((1, dout), lambda i: (0, 0)),
      ],
      out_specs=pl.BlockSpec((_BN, dout), lambda i: (i, 0)),
      out_shape=jax.ShapeDtypeStruct((n, dout), jnp.float32),
  )(agg, x, w_rel, w_root, b)


def _mut_update(h, hprev, midx, w, b):
  n, p = h.shape
  din = hprev.shape[1]

  def body(idx_ref, hprev_hbm, h_hbm, w_ref, b_ref, out_hbm, mut_ref,
           prev_v, cur_v, new_v, sem):
    for i in range(_B):
      idx = idx_ref[i]
      pltpu.async_copy(hprev_hbm.at[pl.ds(idx, 1)],
                       prev_v.at[pl.ds(i, 1)], sem).wait()
      pltpu.async_copy(h_hbm.at[pl.ds(idx, 1)],
                       cur_v.at[pl.ds(i, 1)], sem).wait()
    z = lax.dot_general(prev_v[...], w_ref[...], (((1,), (1,)), ((), ())),
                        preferred_element_type=jnp.float32)
    z = z + b_ref[...] + cur_v[...]
    new_v[...] = z
    mut_ref[...] = z
    for i in range(_B):
      idx = idx_ref[i]
      pltpu.async_copy(new_v.at[pl.ds(i, 1)],
                       out_hbm.at[pl.ds(idx, 1)], sem).wait()

  return pl.pallas_call(
      body,
      in_specs=[
          pl.BlockSpec(memory_space=pltpu.SMEM),
          pl.BlockSpec(memory_space=pltpu.MemorySpace.HBM),
          pl.BlockSpec(memory_space=pltpu.MemorySpace.HBM),
          pl.BlockSpec((p, din), lambda: (0, 0)),
          pl.BlockSpec((1, p), lambda: (0, 0)),
      ],
      out_specs=[
          pl.BlockSpec(memory_space=pltpu.MemorySpace.HBM),
          pl.BlockSpec((_B, p), lambda: (0, 0)),
      ],
      out_shape=[
          jax.ShapeDtypeStruct((n, p), jnp.float32),
          jax.ShapeDtypeStruct((_B, p), jnp.float32),
      ],
      input_output_aliases={2: 0},
      scratch_shapes=[
          pltpu.VMEM((_B, din), jnp.float32),
          pltpu.VMEM((_B, p), jnp.float32),
          pltpu.VMEM((_B, p), jnp.float32),
          pltpu.SemaphoreType.DMA,
      ],
  )(midx, hprev, h, w, b)


def _pool(h, batch2d):
  n, p = h.shape

  def body(b_ref, h_ref, o_ref):
    @pl.when(pl.program_id(0) == 0)
    def _():
      o_ref[...] = jnp.zeros_like(o_ref)

    onehot = (lax.broadcasted_iota(jnp.int32, (_BN, _B), 1) == b_ref[...]
              ).astype(jnp.float32)
    o_ref[...] += lax.dot_general(onehot, h_ref[...], (((0,), (0,)), ((), ())),
                                  preferred_element_type=jnp.float32)

  return pl.pallas_call(
      body,
      grid=(n // _BN,),
      in_specs=[
          pl.BlockSpec((_BN, 1), lambda i: (i, 0)),
          pl.BlockSpec((_BN, p), lambda i: (i, 0)),
      ],
      out_specs=pl.BlockSpec((_B, p), lambda i: (0, 0)),
      out_shape=jax.ShapeDtypeStruct((_B, p), jnp.float32),
  )(batch2d, h)


def _head(g_be, n_be, g_af, n_af, wa, ba, wb, bb):
  def body(bb_ref, g1, n1, g2, n2, wa_ref, ba_ref, wb_ref, o_ref):
    feat = jnp.concatenate([g1[...], n1[...], g2[...], n2[...]], axis=1)
    z = lax.dot_general(feat, wa_ref[...], (((1,), (1,)), ((), ())),
                        preferred_element_type=jnp.float32) + ba_ref[...]
    z = jnp.where(z > 0, z, 0.1 * z)
    o_ref[...] = lax.dot_general(z, wb_ref[...], (((1,), (1,)), ((), ())),
                                 preferred_element_type=jnp.float32) + bb_ref[0]

  return pl.pallas_call(
      body,
      in_specs=[
          pl.BlockSpec(memory_space=pltpu.SMEM),
          pl.BlockSpec((_B, _EMB_P), lambda: (0, 0)),
          pl.BlockSpec((_B, _EMB_P), lambda: (0, 0)),
          pl.BlockSpec((_B, _EMB_P), lambda: (0, 0)),
          pl.BlockSpec((_B, _EMB_P), lambda: (0, 0)),
          pl.BlockSpec((_EMB_P, 4 * _EMB_P), lambda: (0, 0)),
          pl.BlockSpec((1, _EMB_P), lambda: (0, 0)),
          pl.BlockSpec((128, _EMB_P), lambda: (0, 0)),
      ],
      out_specs=pl.BlockSpec((_B, 128), lambda: (0, 0)),
      out_shape=jax.ShapeDtypeStruct((_B, 128), jnp.float32),
  )(bb, g_be, n_be, g_af, n_af, wa, ba, wb)


def _pad2(w, r, c):
  return jnp.pad(w, ((0, r - w.shape[0]), (0, c - w.shape[1])))


def _offidx(idx, count):
  off = jnp.concatenate(
      [jnp.zeros((1,), count.dtype), jnp.cumsum(count)[:-1]])
  return idx + off


def kernel(x_s, edge_index_s, x_s_batch, wide_res_idx, wt_count, x_t,
           edge_index_t, x_t_batch, mut_res_idx, mut_count, W_rel0, b_rel0,
           W_root0, W_rel1, b_rel1, W_root1, W_fc1, b_fc1, W_fc2, b_fc2,
           W_fca, b_fca, W_fcb, b_fcb):
  w_rel0 = _pad2(W_rel0, _EMB_P, _IN_P)
  w_root0 = _pad2(W_root0, _EMB_P, _IN_P)
  w_fc1 = _pad2(W_fc1, _EMB_P, _IN_P)
  w_rel1 = _pad2(W_rel1, _EMB_P, _EMB_P)
  w_root1 = _pad2(W_root1, _EMB_P, _EMB_P)
  w_fc2 = _pad2(W_fc2, _EMB_P, _EMB_P)
  b_rel0p = jnp.pad(b_rel0, (0, _EMB_P - _EMB)).reshape(1, _EMB_P)
  b_rel1p = jnp.pad(b_rel1, (0, _EMB_P - _EMB)).reshape(1, _EMB_P)
  b_fc1p = jnp.pad(b_fc1, (0, _EMB_P - _EMB)).reshape(1, _EMB_P)
  b_fc2p = jnp.pad(b_fc2, (0, _EMB_P - _EMB)).reshape(1, _EMB_P)
  w_fca = jnp.zeros((_EMB_P, 4 * _EMB_P), jnp.float32)
  for j in range(4):
    w_fca = w_fca.at[:_EMB, j * _EMB_P:j * _EMB_P + _EMB].set(
        W_fca[:, j * _EMB:(j + 1) * _EMB])
  b_fcap = jnp.pad(b_fca, (0, _EMB_P - _EMB)).reshape(1, _EMB_P)
  w_fcb = _pad2(W_fcb, 128, _EMB_P)
  b_fcb2 = b_fcb.reshape(1)

  c0 = _IN_P // _CW
  c1 = _EMB_P // _CW

  def one_graph(x, ei, batch, midx):
    xp = jnp.pad(x, ((0, 0), (0, _IN_P - _IN)))
    dst2d = _pad_dst(ei[1])
    src0 = _pad_src(ei[0], c0)
    src1 = _pad_src(ei[0], c1)
    agg0 = _unchunked(_segsum0(xp.reshape(_N * c0, _CW), src0, dst2d), _IN_P)
    h1 = _dense_layer(agg0, xp, w_rel0, w_root0, b_rel0p, relu=True)
    h1, _ = _mut_update(h1, xp, midx, w_fc1, b_fc1p)
    agg1 = _unchunked(_segsum1(h1.reshape(_N * c1, _CW), src1, dst2d), _EMB_P)
    h2 = _dense_layer(agg1, h1, w_rel1, w_root1, b_rel1p, relu=False)
    h2, mut = _mut_update(h2, h1, midx, w_fc2, b_fc2p)
    g = _pool(h2, batch.reshape(_N, 1))
    return g, mut

  widx = _offidx(wide_res_idx, wt_count)
  midx = _offidx(mut_res_idx, mut_count)
  g_be, n_be = one_graph(x_s, edge_index_s, x_s_batch, widx)
  g_af, n_af = one_graph(x_t, edge_index_t, x_t_batch, midx)
  out = _head(g_be, n_be, g_af, n_af, w_fca, b_fcap, w_fcb, b_fcb2)
  return out[:, :1]
